# baseline (speedup 1.0000x reference)
import functools

import jax
import jax.numpy as jnp
from jax import lax
from jax.experimental import pallas as pl
from jax.experimental.pallas import tpu as pltpu
from jax.experimental.pallas import tpu_sc as plsc

_NC = 2
_NS = 16
_NW = _NC * _NS
_B = 8192
_BPW = _B // _NW
_K = 32
_NIDX = 5
_IC = 128
_NCH = _BPW // _IC

_mesh = plsc.VectorSubcoreMesh(core_axis_name="c", subcore_axis_name="s")


@functools.partial(
    pl.kernel,
    mesh=_mesh,
    out_type=[jax.ShapeDtypeStruct((_B, _K), jnp.float32)] * _NIDX,
    scratch_types=[
        pltpu.VMEM((_NIDX * _NCH, _IC), jnp.int32),
        pltpu.VMEM((_NIDX, _BPW, _K), jnp.float32),
        pltpu.SemaphoreType.DMA,
        pltpu.SemaphoreType.DMA,
    ],
    compiler_params=pltpu.CompilerParams(use_tc_tiling_on_sc=False),
)
def _gather5(hs, ls, ts, hcs, tcs, emb_E, emb_R,
             o_hs, o_ls, o_ts, o_hcs, o_tcs,
             idx_v, rows_v, isem, gsem):
    wid = lax.axis_index("s") * _NC + lax.axis_index("c")
    base = wid * _BPW
    srcs = (hs, ls, ts, hcs, tcs)
    tables = (emb_E, emb_R, emb_E, emb_E, emb_E)
    outs = (o_hs, o_ls, o_ts, o_hcs, o_tcs)

    icopies = [
        pltpu.async_copy(
            srcs[i].at[wid],
            idx_v.at[pl.ds(i * _NCH, _NCH)],
            isem,
        )
        for i in range(_NIDX)
    ]
    for c in icopies:
        c.wait()

    gcopies = []
    for i in range(_NIDX):
        for j in range(_NCH):
            gcopies.append(pltpu.async_copy(
                tables[i].at[idx_v.at[i * _NCH + j]],
                rows_v.at[i, pl.ds(j * _IC, _IC)],
                gsem,
            ))
    for c in gcopies:
        c.wait()

    for i in range(_NIDX):
        pltpu.sync_copy(rows_v.at[i], outs[i].at[pl.ds(base, _BPW)])


def kernel(X, emb_E, emb_R):
    half = X.shape[0] // 2
    hs = X[:half, 0].reshape(_NW, _NCH, _IC)
    ls = X[:half, 1].reshape(_NW, _NCH, _IC)
    ts = X[:half, 2].reshape(_NW, _NCH, _IC)
    hcs = X[half:, 0].reshape(_NW, _NCH, _IC)
    tcs = X[half:, 2].reshape(_NW, _NCH, _IC)
    return tuple(_gather5(hs, ls, ts, hcs, tcs, emb_E, emb_R))
